# baseline (speedup 1.0000x reference)
import functools

import jax
import jax.numpy as jnp
from jax import lax
from jax.experimental import pallas as pl
from jax.experimental.pallas import tpu as pltpu
from jax.experimental.pallas import tpu_sc as plsc

N = 10000
E = 320000
D_IN = 128
H1 = 4
D1 = 128
H2 = 1
D2 = 128
HID = 256
NV = 5000

NC = 2
NS = 16
NW = NC * NS
EPW = E // NW
NPS = N // NS

CA = 400
CB = 80
BLK = 2000
CPB = BLK // CB

_MESH = plsc.VectorSubcoreMesh(core_axis_name="c", subcore_axis_name="s")


def _k1_body(x_ref, w1_ref, av_ref, hp0, hp1, hp2, hp3, alpha_ref):
    hp = jnp.dot(x_ref[...], w1_ref[...], preferred_element_type=jnp.float32)
    outs = (hp0, hp1, hp2, hp3)
    for h in range(H1):
        outs[h][...] = hp[:, h * D1:(h + 1) * D1]
    alpha_ref[...] = jnp.dot(hp, av_ref[...], preferred_element_type=jnp.float32)


def _k1(x, W1, Avec):
    blk = 1000
    grid = (N // blk,)
    return pl.pallas_call(
        _k1_body,
        grid=grid,
        in_specs=[
            pl.BlockSpec((blk, D_IN), lambda b: (b, 0)),
            pl.BlockSpec((D_IN, H1 * D1), lambda b: (0, 0)),
            pl.BlockSpec((H1 * D1, 8), lambda b: (0, 0)),
        ],
        out_specs=[pl.BlockSpec((blk, D1), lambda b: (b, 0)) for _ in range(H1)]
        + [pl.BlockSpec((blk, 8), lambda b: (b, 0))],
        out_shape=[jax.ShapeDtypeStruct((N, D1), jnp.float32) for _ in range(H1)]
        + [jax.ShapeDtypeStruct((N, 8), jnp.float32)],
    )(x, W1, Avec)


def _attn_body(heads, src_hbm, dst_hbm, alpha_hbm, ee_hbms, dpart_hbm,
               alpha_v, denom_v, src_b, dst_b, ee_bs):
    wid = lax.axis_index("s") * NC + lax.axis_index("c")
    base = wid * EPW

    pltpu.sync_copy(alpha_hbm, alpha_v)

    @pl.loop(0, N * heads // 16)
    def _zero(i):
        denom_v[pl.ds(i * 16, 16)] = jnp.zeros((16,), jnp.float32)

    @pl.loop(0, EPW // CA)
    def _chunk(ci):
        off = base + ci * CA
        pltpu.sync_copy(src_hbm.at[pl.ds(off, CA)], src_b)
        pltpu.sync_copy(dst_hbm.at[pl.ds(off, CA)], dst_b)

        @pl.loop(0, CA // 16)
        def _e16(i):
            s16 = src_b[pl.ds(i * 16, 16)]
            d16 = dst_b[pl.ds(i * 16, 16)]
            s8 = s16 * 8
            d8 = d16 * 8
            for h in range(heads):
                a_s = plsc.load_gather(alpha_v, [s8 + h])
                a_d = plsc.load_gather(alpha_v, [d8 + (4 + h)])
                e = a_s + a_d
                e = jnp.where(e > 0, e, 0.2 * e)
                ee = jnp.exp(e)
                ee_bs[h][pl.ds(i * 16, 16)] = ee
                plsc.addupdate_scatter(denom_v, [d16 + h * N], ee)

        for h in range(heads):
            pltpu.sync_copy(ee_bs[h], ee_hbms[h].at[pl.ds(off, CA)])

    pltpu.sync_copy(denom_v, dpart_hbm.at[wid])


def _attn(heads, src, dst, alpha_flat):
    def body(src_hbm, dst_hbm, alpha_hbm, *rest):
        ee_hbms = rest[:heads]
        dpart_hbm = rest[heads]
        alpha_v, denom_v, src_b, dst_b = rest[heads + 1:heads + 5]
        ee_bs = rest[heads + 5:]
        _attn_body(heads, src_hbm, dst_hbm, alpha_hbm, ee_hbms, dpart_hbm,
                   alpha_v, denom_v, src_b, dst_b, ee_bs)

    f = pl.kernel(
        body,
        out_type=[jax.ShapeDtypeStruct((E,), jnp.float32) for _ in range(heads)]
        + [jax.ShapeDtypeStruct((NW, N * heads), jnp.float32)],
        mesh=_MESH,
        compiler_params=pltpu.CompilerParams(needs_layout_passes=False),
        scratch_types=[
            pltpu.VMEM((N * 8,), jnp.float32),
            pltpu.VMEM((N * heads,), jnp.float32),
            pltpu.VMEM((CA,), jnp.int32),
            pltpu.VMEM((CA,), jnp.int32),
        ] + [pltpu.VMEM((CA,), jnp.float32) for _ in range(heads)],
    )
    return f(src, dst, alpha_flat)


def _msg_body(heads, src_hbm, dst_hbm, dpart_hbm, ee_hbms, hp_hbms, agg_hbms,
              den_hbm, sh_agg, rows, rows1, redbuf, redout, den_v,
              srcblk, dstblk, eeblk, src_b0, src_b1, dst_b0, dst_b1,
              gsem, ssem):
    cid = lax.axis_index("c")
    sid = lax.axis_index("s")
    wid = sid * NC + cid
    base = wid * EPW
    drain0 = sid * 624
    is_last = sid == NS - 1

    def _slice_copy(src_fn, dst_fn):
        for k in range(7):
            pltpu.sync_copy(src_fn(k * 80, 80), dst_fn(k * 80, 80))
        pltpu.sync_copy(src_fn(560, 64), dst_fn(560, 64))

    NH = N * heads
    cbase = drain0 * heads

    @pl.loop(0, 3 * heads)
    def _redchunk(k):
        off = cbase + k * 208
        for w in range(NW):
            pltpu.sync_copy(dpart_hbm.at[pl.ds(w * NH + off, 208)],
                            redbuf.at[pl.ds(w * 208, 208)])

        @pl.loop(0, 208 // 16)
        def _red(i):
            acc = redbuf[pl.ds(i * 16, 16)]
            for w in range(1, NW):
                acc = acc + redbuf[pl.ds(w * 208 + i * 16, 16)]
            redout[pl.ds(i * 16, 16)] = acc

        pltpu.sync_copy(redout.at[pl.ds(0, 208)], den_hbm.at[pl.ds(off, 208)])

    @pl.when(is_last)
    def _redtail():
        toff = 9984 * heads
        tn = 16 * heads
        for w in range(NW):
            pltpu.sync_copy(dpart_hbm.at[pl.ds(w * NH + toff, tn)],
                            redbuf.at[pl.ds(w * 208, tn)])

        @pl.loop(0, tn // 16)
        def _redt(i):
            acc = redbuf[pl.ds(i * 16, 16)]
            for w in range(1, NW):
                acc = acc + redbuf[pl.ds(w * 208 + i * 16, 16)]
            redout[pl.ds(i * 16, 16)] = acc

        pltpu.sync_copy(redout.at[pl.ds(0, tn)], den_hbm.at[pl.ds(toff, tn)])

    plsc.subcore_barrier()

    for h in range(heads):
        pltpu.sync_copy(den_hbm.at[pl.ds(h * N, N)], den_v)

        @pl.loop(0, CB)
        def _zrow2(i):
            for j in range(D1 // 16):
                rows[i, pl.ds(j * 16, 16)] = jnp.zeros((16,), jnp.float32)

        _slice_copy(lambda o, n: rows.at[pl.ds(0, n)],
                    lambda o, n: sh_agg.at[pl.ds(drain0 + o, n)])

        @pl.when(is_last)
        def _ztail():
            pltpu.sync_copy(rows.at[pl.ds(0, 16)],
                            sh_agg.at[pl.ds(N - 16, 16)])

        plsc.subcore_barrier()

        hp_h = hp_hbms[h]

        def _prep_issue(c, src_bb, rowsbuf):
            for g in range(CB // 16):
                src_bb[pl.ds(g * 16, 16)] = srcblk[pl.ds(c * CB + g * 16, 16)]
            pltpu.async_copy(hp_h.at[src_bb], rowsbuf, gsem)

        def _wait_g(src_bb, rowsbuf):
            pltpu.make_async_copy(hp_h.at[src_bb], rowsbuf, gsem).wait()

        def _wait_s(dst_bb, rowsbuf):
            pltpu.make_async_copy(rowsbuf, sh_agg.at[dst_bb], ssem).wait()

        def _scale_issue(c, dst_bb, rowsbuf):
            @plsc.parallel_loop(0, CB // 16)
            def _scale(g):
                d16 = dstblk[pl.ds(c * CB + g * 16, 16)]
                denv = plsc.load_gather(den_v, [d16])
                wv = eeblk[pl.ds(c * CB + g * 16, 16)] / (denv + 1e-9)
                for l in range(16):
                    s = wv[l]
                    r = g * 16 + l
                    for j in range(D1 // 16):
                        rowsbuf[r, pl.ds(j * 16, 16)] = (
                            rowsbuf[r, pl.ds(j * 16, 16)] * s)

            for g in range(CB // 16):
                dst_bb[pl.ds(g * 16, 16)] = dstblk[pl.ds(c * CB + g * 16, 16)]
            pltpu.async_copy(rowsbuf, sh_agg.at[dst_bb], ssem, add=True)

        @pl.loop(0, EPW // BLK)
        def _block(bi):
            boff = base + bi * BLK
            pltpu.sync_copy(src_hbm.at[pl.ds(boff, BLK)], srcblk)
            pltpu.sync_copy(dst_hbm.at[pl.ds(boff, BLK)], dstblk)
            pltpu.sync_copy(ee_hbms[h].at[pl.ds(boff, BLK)], eeblk)

            pltpu.async_copy(hp_h.at[pl.ds(0, CB)], rows1, ssem)
            _prep_issue(0, src_b0, rows)

            @pl.loop(0, CPB // 2)
            def _pair(i):
                c0 = 2 * i
                _wait_s(dst_b1, rows1)
                _prep_issue(c0 + 1, src_b1, rows1)
                _wait_g(src_b0, rows)
                _scale_issue(c0, dst_b0, rows)
                _wait_g(src_b1, rows1)
                _scale_issue(c0 + 1, dst_b1, rows1)
                _wait_s(dst_b0, rows)
                _prep_issue(c0 + 2, src_b0, rows)

            _wait_s(dst_b1, rows1)
            _wait_g(src_b0, rows)
            _scale_issue(CPB - 1, dst_b0, rows)
            _wait_s(dst_b0, rows)

        plsc.subcore_barrier()
        _slice_copy(lambda o, n: sh_agg.at[pl.ds(drain0 + o, n)],
                    lambda o, n: agg_hbms[h].at[cid, pl.ds(drain0 + o, n)])

        @pl.when(is_last)
        def _dtail():
            pltpu.sync_copy(sh_agg.at[pl.ds(N - 16, 16)],
                            agg_hbms[h].at[cid, pl.ds(N - 16, 16)])

        if h + 1 < heads:
            plsc.subcore_barrier()


def _msg(heads, src, dst, dpart_flat, ees, hps):
    def body(src_hbm, dst_hbm, dpart_hbm, *rest):
        ee_hbms = rest[:heads]
        hp_hbms = rest[heads:2 * heads]
        agg_hbms = rest[2 * heads:3 * heads]
        den_hbm = rest[3 * heads]
        (sh_agg, rows, rows1, redbuf, redout, den_v,
         srcblk, dstblk, eeblk, src_b0, src_b1, dst_b0, dst_b1,
         gsem, ssem) = rest[3 * heads + 1:]
        _msg_body(heads, src_hbm, dst_hbm, dpart_hbm, ee_hbms, hp_hbms,
                  agg_hbms, den_hbm, sh_agg, rows, rows1, redbuf, redout,
                  den_v, srcblk, dstblk, eeblk, src_b0, src_b1, dst_b0,
                  dst_b1, gsem, ssem)

    f = pl.kernel(
        body,
        out_type=[jax.ShapeDtypeStruct((NC, N, D1), jnp.float32)
                  for _ in range(heads)]
        + [jax.ShapeDtypeStruct((N * heads,), jnp.float32)],
        mesh=_MESH,
        compiler_params=pltpu.CompilerParams(needs_layout_passes=False),
        scratch_types=[
            pltpu.VMEM_SHARED((N, D1), jnp.float32),
            pltpu.VMEM((CB, D1), jnp.float32),
            pltpu.VMEM((CB, D1), jnp.float32),
            pltpu.VMEM((NW * 208,), jnp.float32),
            pltpu.VMEM((208,), jnp.float32),
            pltpu.VMEM((N,), jnp.float32),
            pltpu.VMEM((BLK,), jnp.int32),
            pltpu.VMEM((BLK,), jnp.int32),
            pltpu.VMEM((BLK,), jnp.float32),
            pltpu.VMEM((CB,), jnp.int32),
            pltpu.VMEM((CB,), jnp.int32),
            pltpu.VMEM((CB,), jnp.int32),
            pltpu.VMEM((CB,), jnp.int32),
            pltpu.SemaphoreType.DMA,
            pltpu.SemaphoreType.DMA,
        ],
    )
    return f(src, dst, dpart_flat, *ees, *hps)[:heads]


def _k2_body(a0, a1, a2, a3, w2_ref, av2_ref, hp2_ref, alpha2_ref):
    aggs = (a0, a1, a2, a3)
    acc = None
    for h in range(H1):
        h1 = aggs[h][0] + aggs[h][1]
        h1 = jnp.where(h1 > 0, h1, jnp.exp(jnp.minimum(h1, 0.0)) - 1.0)
        part = jnp.dot(h1, w2_ref[h * D1:(h + 1) * D1, :],
                       preferred_element_type=jnp.float32)
        acc = part if acc is None else acc + part
    hp2_ref[...] = acc
    alpha2_ref[...] = jnp.dot(acc, av2_ref[...], preferred_element_type=jnp.float32)


def _k2(aggs, W2, Avec2):
    blk = 1000
    grid = (N // blk,)
    return pl.pallas_call(
        _k2_body,
        grid=grid,
        in_specs=[pl.BlockSpec((NC, blk, D1), lambda b: (0, b, 0))
                  for _ in range(H1)]
        + [
            pl.BlockSpec((H1 * D1, D2), lambda b: (0, 0)),
            pl.BlockSpec((D2, 8), lambda b: (0, 0)),
        ],
        out_specs=[
            pl.BlockSpec((blk, D2), lambda b: (b, 0)),
            pl.BlockSpec((blk, 8), lambda b: (b, 0)),
        ],
        out_shape=[
            jax.ShapeDtypeStruct((N, D2), jnp.float32),
            jax.ShapeDtypeStruct((N, 8), jnp.float32),
        ],
    )(*aggs, W2, Avec2)


def _k3_body(agg_ref, out_ref):
    out_ref[...] = agg_ref[0] + agg_ref[1]


def _k3(agg2):
    blk = 1000
    grid = (N // blk,)
    return pl.pallas_call(
        _k3_body,
        grid=grid,
        in_specs=[
            pl.BlockSpec((NC, blk, D2), lambda b: (0, b, 0)),
        ],
        out_specs=pl.BlockSpec((blk, D2), lambda b: (b, 0)),
        out_shape=jax.ShapeDtypeStruct((N, D2), jnp.float32),
    )(agg2)


def _gather_body(h2_hbm, vidx_hbm, out_hbm, idx0, idx1, rows):
    wid = lax.axis_index("s") * NC + lax.axis_index("c")
    start = jnp.minimum(wid * 160, NV - 160)
    bufs = (idx0, idx1)
    for k in range(2):
        pltpu.sync_copy(vidx_hbm.at[pl.ds(start + k * 80, 80)], bufs[k])
        pltpu.sync_copy(h2_hbm.at[bufs[k]], rows)
        pltpu.sync_copy(rows, out_hbm.at[pl.ds(start + k * 80, 80)])


def _gather(h2, vidx):
    f = pl.kernel(
        _gather_body,
        out_type=jax.ShapeDtypeStruct((NV, D2), jnp.float32),
        mesh=_MESH,
        scratch_types=[
            pltpu.VMEM((80,), jnp.int32),
            pltpu.VMEM((80,), jnp.int32),
            pltpu.VMEM((80, D2), jnp.float32),
        ],
    )
    return f(h2, vidx)


def _k4_body(g_ref, wd1_ref, bd1_ref, wd2_ref, bd2_ref, out_ref):
    hd = jnp.dot(g_ref[...], wd1_ref[...], preferred_element_type=jnp.float32)
    hd = hd + bd1_ref[...]
    hd = jnp.where(hd > 0, hd, 0.3 * hd)
    out_ref[...] = jnp.dot(hd, wd2_ref[...],
                           preferred_element_type=jnp.float32) + bd2_ref[...]


def _k4(g_emb, Wd1, bd1, Wd2, bd2):
    blk = 1000
    grid = (NV // blk,)
    return pl.pallas_call(
        _k4_body,
        grid=grid,
        in_specs=[
            pl.BlockSpec((blk, D2), lambda b: (b, 0)),
            pl.BlockSpec((D2, HID), lambda b: (0, 0)),
            pl.BlockSpec((1, HID), lambda b: (0, 0)),
            pl.BlockSpec((HID, 256), lambda b: (0, 0)),
            pl.BlockSpec((1, 256), lambda b: (0, 0)),
        ],
        out_specs=pl.BlockSpec((blk, 256), lambda b: (b, 0)),
        out_shape=jax.ShapeDtypeStruct((NV, 256), jnp.float32),
    )(g_emb, Wd1, bd1[None, :], Wd2, bd2[None, :])


def kernel(x, edge_index, var_node_index, W1, a1_src, a1_dst, W2, a2_src,
           a2_dst, Wd1, bd1, Wd2, bd2):
    src = edge_index[0].astype(jnp.int32)
    dst = edge_index[1].astype(jnp.int32)
    vidx = var_node_index.astype(jnp.int32)

    def make_avec(a_s, a_d, heads, d):
        cols = []
        for h in range(4):
            if h < heads:
                col = jnp.zeros((heads * d,), jnp.float32).at[h * d:(h + 1) * d].set(a_s[h])
            else:
                col = jnp.zeros((heads * d,), jnp.float32)
            cols.append(col)
        for h in range(4):
            if h < heads:
                col = jnp.zeros((heads * d,), jnp.float32).at[h * d:(h + 1) * d].set(a_d[h])
            else:
                col = jnp.zeros((heads * d,), jnp.float32)
            cols.append(col)
        return jnp.stack(cols, axis=1)

    Avec1 = make_avec(a1_src, a1_dst, H1, D1)
    Avec2 = make_avec(a2_src, a2_dst, H2, D2)

    *hp1s, alpha1 = _k1(x, W1, Avec1)
    out1 = _attn(H1, src, dst, alpha1.reshape(-1))
    ee1s, dpart1 = out1[:H1], out1[H1]
    agg1s = _msg(H1, src, dst, dpart1.reshape(-1), ee1s, hp1s)

    hp2, alpha2 = _k2(agg1s, W2, Avec2)

    out2 = _attn(H2, src, dst, alpha2.reshape(-1))
    ee2s, dpart2 = out2[:H2], out2[H2]
    agg2s = _msg(H2, src, dst, dpart2.reshape(-1), ee2s, [hp2])

    h2 = _k3(agg2s[0])

    g_emb = _gather(h2, vidx)
    return _k4(g_emb, Wd1, bd1, Wd2, bd2)


# speedup vs baseline: 1.0957x; 1.0957x over previous
import functools

import jax
import jax.numpy as jnp
from jax import lax
from jax.experimental import pallas as pl
from jax.experimental.pallas import tpu as pltpu
from jax.experimental.pallas import tpu_sc as plsc

N = 10000
E = 320000
D_IN = 128
H1 = 4
D1 = 128
H2 = 1
D2 = 128
HID = 256
NV = 5000

NC = 2
NS = 16
NW = NC * NS
EPW = E // NW
NPS = N // NS

CA = 400
CB = 80
BLK = 2000
CPB = BLK // CB

_MESH = plsc.VectorSubcoreMesh(core_axis_name="c", subcore_axis_name="s")


def _k1_body(x_ref, w1_ref, av_ref, hp0, hp1, hp2, hp3, alpha_ref):
    hp = jnp.dot(x_ref[...], w1_ref[...], preferred_element_type=jnp.float32)
    outs = (hp0, hp1, hp2, hp3)
    for h in range(H1):
        outs[h][...] = hp[:, h * D1:(h + 1) * D1]
    alpha_ref[...] = jnp.dot(hp, av_ref[...], preferred_element_type=jnp.float32)


def _k1(x, W1, Avec):
    blk = 1000
    grid = (N // blk,)
    return pl.pallas_call(
        _k1_body,
        grid=grid,
        in_specs=[
            pl.BlockSpec((blk, D_IN), lambda b: (b, 0)),
            pl.BlockSpec((D_IN, H1 * D1), lambda b: (0, 0)),
            pl.BlockSpec((H1 * D1, 8), lambda b: (0, 0)),
        ],
        out_specs=[pl.BlockSpec((blk, D1), lambda b: (b, 0)) for _ in range(H1)]
        + [pl.BlockSpec((blk, 8), lambda b: (b, 0))],
        out_shape=[jax.ShapeDtypeStruct((N, D1), jnp.float32) for _ in range(H1)]
        + [jax.ShapeDtypeStruct((N, 8), jnp.float32)],
    )(x, W1, Avec)


def _attn_body(heads, src_hbm, dst_hbm, alpha_hbm, ee_hbms, dpart_hbm,
               alpha_v, denom_v, src_b, dst_b, ee_bs):
    wid = lax.axis_index("s") * NC + lax.axis_index("c")
    base = wid * EPW

    pltpu.sync_copy(alpha_hbm, alpha_v)

    @pl.loop(0, N * heads // 16)
    def _zero(i):
        denom_v[pl.ds(i * 16, 16)] = jnp.zeros((16,), jnp.float32)

    @pl.loop(0, EPW // CA)
    def _chunk(ci):
        off = base + ci * CA
        pltpu.sync_copy(src_hbm.at[pl.ds(off, CA)], src_b)
        pltpu.sync_copy(dst_hbm.at[pl.ds(off, CA)], dst_b)

        @plsc.parallel_loop(0, CA // 16, unroll=2)
        def _e16(i):
            s16 = src_b[pl.ds(i * 16, 16)]
            d16 = dst_b[pl.ds(i * 16, 16)]
            s8 = s16 * 8
            d8 = d16 * 8
            for h in range(heads):
                a_s = plsc.load_gather(alpha_v, [s8 + h])
                a_d = plsc.load_gather(alpha_v, [d8 + (4 + h)])
                e = a_s + a_d
                e = jnp.where(e > 0, e, 0.2 * e)
                ee = jnp.exp(e)
                ee_bs[h][pl.ds(i * 16, 16)] = ee
                plsc.addupdate_scatter(denom_v, [d16 + h * N], ee)

        for h in range(heads):
            pltpu.sync_copy(ee_bs[h], ee_hbms[h].at[pl.ds(off, CA)])

    pltpu.sync_copy(denom_v, dpart_hbm.at[wid])


def _attn(heads, src, dst, alpha_flat):
    def body(src_hbm, dst_hbm, alpha_hbm, *rest):
        ee_hbms = rest[:heads]
        dpart_hbm = rest[heads]
        alpha_v, denom_v, src_b, dst_b = rest[heads + 1:heads + 5]
        ee_bs = rest[heads + 5:]
        _attn_body(heads, src_hbm, dst_hbm, alpha_hbm, ee_hbms, dpart_hbm,
                   alpha_v, denom_v, src_b, dst_b, ee_bs)

    f = pl.kernel(
        body,
        out_type=[jax.ShapeDtypeStruct((E,), jnp.float32) for _ in range(heads)]
        + [jax.ShapeDtypeStruct((NW, N * heads), jnp.float32)],
        mesh=_MESH,
        compiler_params=pltpu.CompilerParams(needs_layout_passes=False),
        scratch_types=[
            pltpu.VMEM((N * 8,), jnp.float32),
            pltpu.VMEM((N * heads,), jnp.float32),
            pltpu.VMEM((CA,), jnp.int32),
            pltpu.VMEM((CA,), jnp.int32),
        ] + [pltpu.VMEM((CA,), jnp.float32) for _ in range(heads)],
    )
    return f(src, dst, alpha_flat)


def _msg_body(heads, src_hbm, dst_hbm, dpart_hbm, ee_hbms, hp_hbms, agg_hbms,
              den_hbm, sh_agg, rows, rows1, redbuf, redout, den_v,
              srcblk, dstblk, eeblk, src_b0, src_b1, dst_b0, dst_b1,
              gsem, ssem):
    cid = lax.axis_index("c")
    sid = lax.axis_index("s")
    wid = sid * NC + cid
    base = wid * EPW
    drain0 = sid * 624
    is_last = sid == NS - 1

    def _slice_copy(src_fn, dst_fn):
        for k in range(7):
            pltpu.sync_copy(src_fn(k * 80, 80), dst_fn(k * 80, 80))
        pltpu.sync_copy(src_fn(560, 64), dst_fn(560, 64))

    NH = N * heads
    cbase = drain0 * heads

    @pl.loop(0, 3 * heads)
    def _redchunk(k):
        off = cbase + k * 208
        for w in range(NW):
            pltpu.sync_copy(dpart_hbm.at[pl.ds(w * NH + off, 208)],
                            redbuf.at[pl.ds(w * 208, 208)])

        @pl.loop(0, 208 // 16)
        def _red(i):
            acc = redbuf[pl.ds(i * 16, 16)]
            for w in range(1, NW):
                acc = acc + redbuf[pl.ds(w * 208 + i * 16, 16)]
            redout[pl.ds(i * 16, 16)] = acc

        pltpu.sync_copy(redout.at[pl.ds(0, 208)], den_hbm.at[pl.ds(off, 208)])

    @pl.when(is_last)
    def _redtail():
        toff = 9984 * heads
        tn = 16 * heads
        for w in range(NW):
            pltpu.sync_copy(dpart_hbm.at[pl.ds(w * NH + toff, tn)],
                            redbuf.at[pl.ds(w * 208, tn)])

        @pl.loop(0, tn // 16)
        def _redt(i):
            acc = redbuf[pl.ds(i * 16, 16)]
            for w in range(1, NW):
                acc = acc + redbuf[pl.ds(w * 208 + i * 16, 16)]
            redout[pl.ds(i * 16, 16)] = acc

        pltpu.sync_copy(redout.at[pl.ds(0, tn)], den_hbm.at[pl.ds(toff, tn)])

    plsc.subcore_barrier()

    for h in range(heads):
        pltpu.sync_copy(den_hbm.at[pl.ds(h * N, N)], den_v)

        @pl.loop(0, CB)
        def _zrow2(i):
            for j in range(D1 // 16):
                rows[i, pl.ds(j * 16, 16)] = jnp.zeros((16,), jnp.float32)

        _slice_copy(lambda o, n: rows.at[pl.ds(0, n)],
                    lambda o, n: sh_agg.at[pl.ds(drain0 + o, n)])

        @pl.when(is_last)
        def _ztail():
            pltpu.sync_copy(rows.at[pl.ds(0, 16)],
                            sh_agg.at[pl.ds(N - 16, 16)])

        plsc.subcore_barrier()

        hp_h = hp_hbms[h]

        def _prep_issue(c, src_bb, rowsbuf):
            for g in range(CB // 16):
                src_bb[pl.ds(g * 16, 16)] = srcblk[pl.ds(c * CB + g * 16, 16)]
            pltpu.async_copy(hp_h.at[src_bb], rowsbuf, gsem)

        def _consume(c, src_bb, dst_bb, rowsbuf):
            pltpu.make_async_copy(hp_h.at[src_bb], rowsbuf, gsem).wait()

            @plsc.parallel_loop(0, CB // 16)
            def _scale(g):
                d16 = dstblk[pl.ds(c * CB + g * 16, 16)]
                denv = plsc.load_gather(den_v, [d16])
                wv = eeblk[pl.ds(c * CB + g * 16, 16)] / (denv + 1e-9)
                for l in range(16):
                    s = wv[l]
                    r = g * 16 + l
                    for j in range(D1 // 16):
                        rowsbuf[r, pl.ds(j * 16, 16)] = (
                            rowsbuf[r, pl.ds(j * 16, 16)] * s)

            for g in range(CB // 16):
                dst_bb[pl.ds(g * 16, 16)] = dstblk[pl.ds(c * CB + g * 16, 16)]
            pltpu.sync_copy(rowsbuf, sh_agg.at[dst_bb], add=True)

        @pl.loop(0, EPW // BLK)
        def _block(bi):
            boff = base + bi * BLK
            pltpu.sync_copy(src_hbm.at[pl.ds(boff, BLK)], srcblk)
            pltpu.sync_copy(dst_hbm.at[pl.ds(boff, BLK)], dstblk)
            pltpu.sync_copy(ee_hbms[h].at[pl.ds(boff, BLK)], eeblk)

            _prep_issue(0, src_b0, rows)

            @pl.loop(0, CPB // 2)
            def _pair(i):
                c0 = 2 * i
                _prep_issue(c0 + 1, src_b1, rows1)
                _consume(c0, src_b0, dst_b0, rows)
                _prep_issue(c0 + 2, src_b0, rows)
                _consume(c0 + 1, src_b1, dst_b1, rows1)

            _consume(CPB - 1, src_b0, dst_b0, rows)

        plsc.subcore_barrier()
        _slice_copy(lambda o, n: sh_agg.at[pl.ds(drain0 + o, n)],
                    lambda o, n: agg_hbms[h].at[cid, pl.ds(drain0 + o, n)])

        @pl.when(is_last)
        def _dtail():
            pltpu.sync_copy(sh_agg.at[pl.ds(N - 16, 16)],
                            agg_hbms[h].at[cid, pl.ds(N - 16, 16)])

        if h + 1 < heads:
            plsc.subcore_barrier()


def _msg(heads, src, dst, dpart_flat, ees, hps):
    def body(src_hbm, dst_hbm, dpart_hbm, *rest):
        ee_hbms = rest[:heads]
        hp_hbms = rest[heads:2 * heads]
        agg_hbms = rest[2 * heads:3 * heads]
        den_hbm = rest[3 * heads]
        (sh_agg, rows, rows1, redbuf, redout, den_v,
         srcblk, dstblk, eeblk, src_b0, src_b1, dst_b0, dst_b1,
         gsem, ssem) = rest[3 * heads + 1:]
        _msg_body(heads, src_hbm, dst_hbm, dpart_hbm, ee_hbms, hp_hbms,
                  agg_hbms, den_hbm, sh_agg, rows, rows1, redbuf, redout,
                  den_v, srcblk, dstblk, eeblk, src_b0, src_b1, dst_b0,
                  dst_b1, gsem, ssem)

    f = pl.kernel(
        body,
        out_type=[jax.ShapeDtypeStruct((NC, N, D1), jnp.float32)
                  for _ in range(heads)]
        + [jax.ShapeDtypeStruct((N * heads,), jnp.float32)],
        mesh=_MESH,
        compiler_params=pltpu.CompilerParams(needs_layout_passes=False),
        scratch_types=[
            pltpu.VMEM_SHARED((N, D1), jnp.float32),
            pltpu.VMEM((CB, D1), jnp.float32),
            pltpu.VMEM((CB, D1), jnp.float32),
            pltpu.VMEM((NW * 208,), jnp.float32),
            pltpu.VMEM((208,), jnp.float32),
            pltpu.VMEM((N,), jnp.float32),
            pltpu.VMEM((BLK,), jnp.int32),
            pltpu.VMEM((BLK,), jnp.int32),
            pltpu.VMEM((BLK,), jnp.float32),
            pltpu.VMEM((CB,), jnp.int32),
            pltpu.VMEM((CB,), jnp.int32),
            pltpu.VMEM((CB,), jnp.int32),
            pltpu.VMEM((CB,), jnp.int32),
            pltpu.SemaphoreType.DMA,
            pltpu.SemaphoreType.DMA,
        ],
    )
    return f(src, dst, dpart_flat, *ees, *hps)[:heads]


def _k2_body(a0, a1, a2, a3, w2_ref, av2_ref, hp2_ref, alpha2_ref):
    aggs = (a0, a1, a2, a3)
    acc = None
    for h in range(H1):
        h1 = aggs[h][0] + aggs[h][1]
        h1 = jnp.where(h1 > 0, h1, jnp.exp(jnp.minimum(h1, 0.0)) - 1.0)
        part = jnp.dot(h1, w2_ref[h * D1:(h + 1) * D1, :],
                       preferred_element_type=jnp.float32)
        acc = part if acc is None else acc + part
    hp2_ref[...] = acc
    alpha2_ref[...] = jnp.dot(acc, av2_ref[...], preferred_element_type=jnp.float32)


def _k2(aggs, W2, Avec2):
    blk = 1000
    grid = (N // blk,)
    return pl.pallas_call(
        _k2_body,
        grid=grid,
        in_specs=[pl.BlockSpec((NC, blk, D1), lambda b: (0, b, 0))
                  for _ in range(H1)]
        + [
            pl.BlockSpec((H1 * D1, D2), lambda b: (0, 0)),
            pl.BlockSpec((D2, 8), lambda b: (0, 0)),
        ],
        out_specs=[
            pl.BlockSpec((blk, D2), lambda b: (b, 0)),
            pl.BlockSpec((blk, 8), lambda b: (b, 0)),
        ],
        out_shape=[
            jax.ShapeDtypeStruct((N, D2), jnp.float32),
            jax.ShapeDtypeStruct((N, 8), jnp.float32),
        ],
    )(*aggs, W2, Avec2)


def _k3_body(agg_ref, out_ref):
    out_ref[...] = agg_ref[0] + agg_ref[1]


def _k3(agg2):
    blk = 1000
    grid = (N // blk,)
    return pl.pallas_call(
        _k3_body,
        grid=grid,
        in_specs=[
            pl.BlockSpec((NC, blk, D2), lambda b: (0, b, 0)),
        ],
        out_specs=pl.BlockSpec((blk, D2), lambda b: (b, 0)),
        out_shape=jax.ShapeDtypeStruct((N, D2), jnp.float32),
    )(agg2)


def _gather_body(h2_hbm, vidx_hbm, out_hbm, idx0, idx1, rows):
    wid = lax.axis_index("s") * NC + lax.axis_index("c")
    start = jnp.minimum(wid * 160, NV - 160)
    bufs = (idx0, idx1)
    for k in range(2):
        pltpu.sync_copy(vidx_hbm.at[pl.ds(start + k * 80, 80)], bufs[k])
        pltpu.sync_copy(h2_hbm.at[bufs[k]], rows)
        pltpu.sync_copy(rows, out_hbm.at[pl.ds(start + k * 80, 80)])


def _gather(h2, vidx):
    f = pl.kernel(
        _gather_body,
        out_type=jax.ShapeDtypeStruct((NV, D2), jnp.float32),
        mesh=_MESH,
        scratch_types=[
            pltpu.VMEM((80,), jnp.int32),
            pltpu.VMEM((80,), jnp.int32),
            pltpu.VMEM((80, D2), jnp.float32),
        ],
    )
    return f(h2, vidx)


def _k4_body(g_ref, wd1_ref, bd1_ref, wd2_ref, bd2_ref, out_ref):
    hd = jnp.dot(g_ref[...], wd1_ref[...], preferred_element_type=jnp.float32)
    hd = hd + bd1_ref[...]
    hd = jnp.where(hd > 0, hd, 0.3 * hd)
    out_ref[...] = jnp.dot(hd, wd2_ref[...],
                           preferred_element_type=jnp.float32) + bd2_ref[...]


def _k4(g_emb, Wd1, bd1, Wd2, bd2):
    blk = 1000
    grid = (NV // blk,)
    return pl.pallas_call(
        _k4_body,
        grid=grid,
        in_specs=[
            pl.BlockSpec((blk, D2), lambda b: (b, 0)),
            pl.BlockSpec((D2, HID), lambda b: (0, 0)),
            pl.BlockSpec((1, HID), lambda b: (0, 0)),
            pl.BlockSpec((HID, 256), lambda b: (0, 0)),
            pl.BlockSpec((1, 256), lambda b: (0, 0)),
        ],
        out_specs=pl.BlockSpec((blk, 256), lambda b: (b, 0)),
        out_shape=jax.ShapeDtypeStruct((NV, 256), jnp.float32),
    )(g_emb, Wd1, bd1[None, :], Wd2, bd2[None, :])


def kernel(x, edge_index, var_node_index, W1, a1_src, a1_dst, W2, a2_src,
           a2_dst, Wd1, bd1, Wd2, bd2):
    src = edge_index[0].astype(jnp.int32)
    dst = edge_index[1].astype(jnp.int32)
    vidx = var_node_index.astype(jnp.int32)

    def make_avec(a_s, a_d, heads, d):
        cols = []
        for h in range(4):
            if h < heads:
                col = jnp.zeros((heads * d,), jnp.float32).at[h * d:(h + 1) * d].set(a_s[h])
            else:
                col = jnp.zeros((heads * d,), jnp.float32)
            cols.append(col)
        for h in range(4):
            if h < heads:
                col = jnp.zeros((heads * d,), jnp.float32).at[h * d:(h + 1) * d].set(a_d[h])
            else:
                col = jnp.zeros((heads * d,), jnp.float32)
            cols.append(col)
        return jnp.stack(cols, axis=1)

    Avec1 = make_avec(a1_src, a1_dst, H1, D1)
    Avec2 = make_avec(a2_src, a2_dst, H2, D2)

    *hp1s, alpha1 = _k1(x, W1, Avec1)
    out1 = _attn(H1, src, dst, alpha1.reshape(-1))
    ee1s, dpart1 = out1[:H1], out1[H1]
    agg1s = _msg(H1, src, dst, dpart1.reshape(-1), ee1s, hp1s)

    hp2, alpha2 = _k2(agg1s, W2, Avec2)

    out2 = _attn(H2, src, dst, alpha2.reshape(-1))
    ee2s, dpart2 = out2[:H2], out2[H2]
    agg2s = _msg(H2, src, dst, dpart2.reshape(-1), ee2s, [hp2])

    h2 = _k3(agg2s[0])

    g_emb = _gather(h2, vidx)
    return _k4(g_emb, Wd1, bd1, Wd2, bd2)
